# Initial kernel scaffold; baseline (speedup 1.0000x reference)
import jax
import jax.numpy as jnp
from jax.experimental import pallas as pl


def kernel(x, params, edge_index, batch):
    raise NotImplementedError("write your pallas kernel here")


# speedup vs baseline: 9.0529x; 9.0529x over previous
import functools

import jax
import jax.numpy as jnp
from jax import lax
from jax.experimental import pallas as pl
from jax.experimental.pallas import tpu as pltpu
from jax.experimental.pallas import tpu_sc as plsc

IN_DIM = 128
HID = 128
HEADS = 8
DH = HID // HEADS
LAYERS = 3
G = 64

NCORE = 2
NSUB = 16
NW = NCORE * NSUB
CHUNK = 128


def _layer_norm(x, g, b, eps=1e-5):
    m = jnp.mean(x, axis=-1, keepdims=True)
    v = jnp.mean((x - m) ** 2, axis=-1, keepdims=True)
    return (x - m) * lax.rsqrt(v + eps) * g + b


def _sc_deg_body(n_pad, rpt, e_chunks, b_chunks,
                 dst_hbm, batch_hbm, ones_hbm, zrow_hbm, zcnt_hbm,
                 deg_out, cnt_out,
                 idx_v, idxb_v, ones_v, shared_deg, shared_cnt):
    c = lax.axis_index("c")
    s = lax.axis_index("s")
    pltpu.sync_copy(zrow_hbm.at[pl.ds(s * rpt, rpt)],
                    shared_deg.at[pl.ds(s * rpt, rpt)])

    @pl.when(s == 0)
    def _():
        pltpu.sync_copy(zcnt_hbm, shared_cnt)

    pltpu.sync_copy(ones_hbm, ones_v)
    plsc.subcore_barrier()

    ebase = (c * NSUB + s) * e_chunks * CHUNK

    def echunk(j, carry):
        pltpu.sync_copy(dst_hbm.at[pl.ds(ebase + j * CHUNK, CHUNK)], idx_v)
        pltpu.sync_copy(ones_v, shared_deg.at[idx_v], add=True)
        return carry

    lax.fori_loop(0, e_chunks, echunk, 0)

    nbase = (c * NSUB + s) * b_chunks * 64

    def bchunk(j, carry):
        pltpu.sync_copy(batch_hbm.at[pl.ds(nbase + j * 64, 64)], idxb_v)
        pltpu.sync_copy(ones_v.at[pl.ds(0, 64)], shared_cnt.at[idxb_v],
                        add=True)
        return carry

    lax.fori_loop(0, b_chunks, bchunk, 0)
    plsc.subcore_barrier()

    pltpu.sync_copy(shared_deg.at[pl.ds(s * rpt, rpt)],
                    deg_out.at[c, pl.ds(s * rpt, rpt)])

    @pl.when(s == 0)
    def _():
        pltpu.sync_copy(shared_cnt, cnt_out.at[c])


def _sc_degrees(dst_pad, batch_pad, n_pad, rpt, e_chunks, b_chunks):
    mesh = plsc.VectorSubcoreMesh(core_axis_name="c", subcore_axis_name="s")
    ones = jnp.ones((CHUNK, 16), jnp.float32)
    zrow = jnp.zeros((n_pad, 16), jnp.float32)
    zcnt = jnp.zeros((G + 16, 16), jnp.float32)
    body = functools.partial(_sc_deg_body, n_pad, rpt, e_chunks, b_chunks)
    return pl.kernel(
        body,
        out_type=[
            jax.ShapeDtypeStruct((NCORE, n_pad, 16), jnp.float32),
            jax.ShapeDtypeStruct((NCORE, G + 16, 16), jnp.float32),
        ],
        mesh=mesh,
        scratch_types=[
            pltpu.VMEM((CHUNK,), jnp.int32),
            pltpu.VMEM((64,), jnp.int32),
            pltpu.VMEM((CHUNK, 16), jnp.float32),
            pltpu.VMEM_SHARED((n_pad, 16), jnp.float32),
            pltpu.VMEM_SHARED((G + 16, 16), jnp.float32),
        ],
    )(dst_pad, batch_pad, ones, zrow, zcnt)


def _sc_agg_body(n_pad, rpt, e_chunks,
                 mp_hbm, src_hbm, dst_hbm, zeros_hbm, agg_out,
                 idx_s, idx_d, rows_v, sem, shared_acc):
    c = lax.axis_index("c")
    s = lax.axis_index("s")
    pltpu.sync_copy(zeros_hbm.at[pl.ds(s * rpt, rpt)],
                    shared_acc.at[pl.ds(s * rpt, rpt)])
    plsc.subcore_barrier()

    ebase = (c * NSUB + s) * e_chunks * CHUNK

    def chunk(j, carry):
        b = ebase + j * CHUNK
        pltpu.sync_copy(src_hbm.at[pl.ds(b, CHUNK)], idx_s)
        pltpu.sync_copy(dst_hbm.at[pl.ds(b, CHUNK)], idx_d)
        pltpu.async_copy(mp_hbm.at[idx_s], rows_v, sem).wait()
        pltpu.sync_copy(rows_v, shared_acc.at[idx_d], add=True)
        return carry

    lax.fori_loop(0, e_chunks, chunk, 0)
    plsc.subcore_barrier()
    pltpu.sync_copy(shared_acc.at[pl.ds(s * rpt, rpt)],
                    agg_out.at[c, pl.ds(s * rpt, rpt)])


def _sc_aggregate(mp, src_pad, dst_pad, zeros_hbm, n_pad, rpt, e_chunks):
    mesh = plsc.VectorSubcoreMesh(core_axis_name="c", subcore_axis_name="s")
    body = functools.partial(_sc_agg_body, n_pad, rpt, e_chunks)
    return pl.kernel(
        body,
        out_type=jax.ShapeDtypeStruct((NCORE, n_pad, HID), jnp.float32),
        mesh=mesh,
        scratch_types=[
            pltpu.VMEM((CHUNK,), jnp.int32),
            pltpu.VMEM((CHUNK,), jnp.int32),
            pltpu.VMEM((CHUNK, HID), jnp.float32),
            pltpu.SemaphoreType.DMA,
            pltpu.VMEM_SHARED((n_pad, HID), jnp.float32),
        ],
    )(mp, src_pad, dst_pad, zeros_hbm)


ROWS = 1000


def _tc_input_body(x_r, w_r, b_r, g_r, be_r, deg_r, w0_r, h0_r, mp_r):
    h = jnp.dot(x_r[...], w_r[...].T, preferred_element_type=jnp.float32)
    h = _layer_norm(h + b_r[...], g_r[...], be_r[...])
    h = jnp.maximum(h, 0.0)
    h0_r[...] = h
    deg = deg_r[0, :, 0:1] + deg_r[1, :, 0:1] + 1.0
    dinv = lax.rsqrt(deg)
    mp_r[...] = jnp.dot(h, w0_r[...].T,
                        preferred_element_type=jnp.float32) * dinv


def _tc_input(x, p, deg, n):
    grid = (n // ROWS,)
    return pl.pallas_call(
        _tc_input_body,
        grid=grid,
        in_specs=[
            pl.BlockSpec((ROWS, IN_DIM), lambda i: (i, 0)),
            pl.BlockSpec((HID, IN_DIM), lambda i: (0, 0)),
            pl.BlockSpec((1, HID), lambda i: (0, 0)),
            pl.BlockSpec((1, HID), lambda i: (0, 0)),
            pl.BlockSpec((1, HID), lambda i: (0, 0)),
            pl.BlockSpec((NCORE, ROWS, 16), lambda i: (0, i, 0)),
            pl.BlockSpec((HID, HID), lambda i: (0, 0)),
        ],
        out_specs=[
            pl.BlockSpec((ROWS, HID), lambda i: (i, 0)),
            pl.BlockSpec((ROWS, HID), lambda i: (i, 0)),
        ],
        out_shape=[
            jax.ShapeDtypeStruct((n, HID), jnp.float32),
            jax.ShapeDtypeStruct((n, HID), jnp.float32),
        ],
    )(x, p['in_w'], p['in_b'].reshape(1, -1), p['in_ln_g'].reshape(1, -1),
      p['in_ln_b'].reshape(1, -1), deg, p['conv_w'][0])


def _tc_layer_body(residual, qkv_mode,
                   agg_r, mp_r, deg_r, hres_r, b_r, g_r, be_r, wn_r, bn_r,
                   hout_r, mout_r):
    deg = deg_r[0, :, 0:1] + deg_r[1, :, 0:1] + 1.0
    dinv = lax.rsqrt(deg)
    pre = (agg_r[0] + agg_r[1] + mp_r[...]) * dinv + b_r[...]
    hh = jnp.maximum(_layer_norm(pre, g_r[...], be_r[...]), 0.0)
    if residual:
        hh = hh + hres_r[...]
    hout_r[...] = hh
    z = jnp.dot(hh, wn_r[...].T, preferred_element_type=jnp.float32)
    if qkv_mode:
        mout_r[...] = z + bn_r[...]
    else:
        mout_r[...] = z * dinv


def _tc_layer(agg, mp, deg, hres, b, g, be, wn, bn, n, residual, qkv_mode):
    grid = (n // ROWS,)
    wdim = wn.shape[0]
    body = functools.partial(_tc_layer_body, residual, qkv_mode)
    return pl.pallas_call(
        body,
        grid=grid,
        in_specs=[
            pl.BlockSpec((NCORE, ROWS, HID), lambda i: (0, i, 0)),
            pl.BlockSpec((ROWS, HID), lambda i: (i, 0)),
            pl.BlockSpec((NCORE, ROWS, 16), lambda i: (0, i, 0)),
            pl.BlockSpec((ROWS, HID), lambda i: (i, 0)),
            pl.BlockSpec((1, HID), lambda i: (0, 0)),
            pl.BlockSpec((1, HID), lambda i: (0, 0)),
            pl.BlockSpec((1, HID), lambda i: (0, 0)),
            pl.BlockSpec((wdim, HID), lambda i: (0, 0)),
            pl.BlockSpec((1, wdim), lambda i: (0, 0)),
        ],
        out_specs=[
            pl.BlockSpec((ROWS, HID), lambda i: (i, 0)),
            pl.BlockSpec((ROWS, wdim), lambda i: (i, 0)),
        ],
        out_shape=[
            jax.ShapeDtypeStruct((n, HID), jnp.float32),
            jax.ShapeDtypeStruct((n, wdim), jnp.float32),
        ],
    )(agg, mp, deg, hres, b.reshape(1, -1), g.reshape(1, -1),
      be.reshape(1, -1), wn, bn.reshape(1, -1))


def _tc_attn_body(offs_r, cnt_r, qkv_r, h_r, wo_r, bo_r, g_r, be_r, out_r):
    gi = pl.program_id(0)
    off = offs_r[gi]
    n = cnt_r[gi]
    off_al = (off // 8) * 8
    d = off - off_al
    nt = (n + d + 127) // 128
    NEG = jnp.float32(-1e30)
    scale = jnp.float32(1.0 / (DH ** 0.5))

    def row_step(rt, row_carry):
        sum_acc, max_acc = row_carry
        r0 = off_al + rt * 128
        q = qkv_r[pl.ds(r0, 128), 0:HID] * scale

        def col_step(ct, cc):
            m_c, l_c, o_c = cc
            c0 = off_al + ct * 128
            k = qkv_r[pl.ds(c0, 128), HID:2 * HID]
            v = qkv_r[pl.ds(c0, 128), 2 * HID:3 * HID]
            colid = ct * 128 + lax.broadcasted_iota(jnp.int32, (1, 128), 1)
            cmask = (colid >= d) & (colid < n + d)
            m_cols, l_cols, o_cols = [], [], []
            for h in range(HEADS):
                sl = slice(h * DH, (h + 1) * DH)
                sc = lax.dot_general(q[:, sl], k[:, sl],
                                     (((1,), (1,)), ((), ())),
                                     preferred_element_type=jnp.float32)
                sc = jnp.where(cmask, sc, NEG)
                m_old = m_c[:, h:h + 1]
                m_new = jnp.maximum(m_old, jnp.max(sc, axis=1, keepdims=True))
                pexp = jnp.exp(sc - m_new)
                alpha = jnp.exp(m_old - m_new)
                l_cols.append(l_c[:, h:h + 1] * alpha
                              + jnp.sum(pexp, axis=1, keepdims=True))
                o_cols.append(o_c[:, sl] * alpha
                              + jnp.dot(pexp, v[:, sl],
                                        preferred_element_type=jnp.float32))
                m_cols.append(m_new)
            return (jnp.concatenate(m_cols, axis=1),
                    jnp.concatenate(l_cols, axis=1),
                    jnp.concatenate(o_cols, axis=1))

        m0 = jnp.full((128, HEADS), NEG, jnp.float32)
        l0 = jnp.zeros((128, HEADS), jnp.float32)
        o0 = jnp.zeros((128, HID), jnp.float32)
        m_c, l_c, o_c = lax.fori_loop(0, nt, col_step, (m0, l0, o0))
        o = jnp.concatenate(
            [o_c[:, h * DH:(h + 1) * DH]
             / jnp.maximum(l_c[:, h:h + 1], jnp.float32(1e-30))
             for h in range(HEADS)], axis=1)
        a = jnp.dot(o, wo_r[...].T,
                    preferred_element_type=jnp.float32) + bo_r[...]
        y = _layer_norm(a + h_r[pl.ds(r0, 128), :], g_r[...], be_r[...])
        rowid = rt * 128 + lax.broadcasted_iota(jnp.int32, (128, 1), 0)
        rmask = (rowid >= d) & (rowid < n + d)
        sum_acc = sum_acc + jnp.sum(jnp.where(rmask, y, 0.0), axis=0,
                                    keepdims=True)
        max_acc = jnp.maximum(
            max_acc,
            jnp.max(jnp.where(rmask, y, -jnp.inf), axis=0, keepdims=True))
        return (sum_acc, max_acc)

    s0 = jnp.zeros((1, HID), jnp.float32)
    mx0 = jnp.full((1, HID), -jnp.inf, jnp.float32)
    sum_acc, max_acc = lax.fori_loop(0, nt, row_step, (s0, mx0))
    mean = sum_acc / jnp.maximum(n.astype(jnp.float32), 1.0)
    out_r[...] = jnp.concatenate([mean, max_acc, sum_acc],
                                 axis=1).reshape(1, 1, 3 * HID)


def _tc_attn(offs, cnt, qkv_pad, h_pad, p):
    npad = qkv_pad.shape[0]
    return pl.pallas_call(
        _tc_attn_body,
        grid=(G,),
        in_specs=[
            pl.BlockSpec(memory_space=pltpu.SMEM),
            pl.BlockSpec(memory_space=pltpu.SMEM),
            pl.BlockSpec((npad, 3 * HID), lambda i: (0, 0)),
            pl.BlockSpec((npad, HID), lambda i: (0, 0)),
            pl.BlockSpec((HID, HID), lambda i: (0, 0)),
            pl.BlockSpec((1, HID), lambda i: (0, 0)),
            pl.BlockSpec((1, HID), lambda i: (0, 0)),
            pl.BlockSpec((1, HID), lambda i: (0, 0)),
        ],
        out_specs=pl.BlockSpec((1, 1, 3 * HID), lambda i: (i, 0, 0)),
        out_shape=jax.ShapeDtypeStruct((G, 1, 3 * HID), jnp.float32),
    )(offs, cnt, qkv_pad, h_pad, p['attn_out_w'],
      p['attn_out_b'].reshape(1, -1), p['attn_ln_g'].reshape(1, -1),
      p['attn_ln_b'].reshape(1, -1))


def _tc_cls_body(x_r, w1_r, b1_r, g_r, be_r, w2_r, b2_r, w3_r, b3_r, out_r):
    z = jnp.dot(x_r[...], w1_r[...].T,
                preferred_element_type=jnp.float32) + b1_r[...]
    z = jnp.maximum(_layer_norm(z, g_r[...], be_r[...]), 0.0)
    z = jnp.maximum(jnp.dot(z, w2_r[...].T,
                            preferred_element_type=jnp.float32) + b2_r[...],
                    0.0)
    out_r[...] = jnp.dot(z, w3_r[...].T,
                         preferred_element_type=jnp.float32) + b3_r[...]


def _tc_cls(pooled, p):
    w2 = jnp.zeros((HID, HID), jnp.float32).at[:HID // 2, :].set(p['cls_w2'])
    b2 = jnp.zeros((HID,), jnp.float32).at[:HID // 2].set(p['cls_b2'])
    w3 = jnp.zeros((HID, HID), jnp.float32).at[:2, :HID // 2].set(p['cls_w3'])
    b3 = jnp.zeros((HID,), jnp.float32).at[:2].set(p['cls_b3'])
    full = lambda i: (0, 0)
    out = pl.pallas_call(
        _tc_cls_body,
        grid=(1,),
        in_specs=[
            pl.BlockSpec((G, 3 * HID), full),
            pl.BlockSpec((HID, 3 * HID), full),
            pl.BlockSpec((1, HID), full),
            pl.BlockSpec((1, HID), full),
            pl.BlockSpec((1, HID), full),
            pl.BlockSpec((HID, HID), full),
            pl.BlockSpec((1, HID), full),
            pl.BlockSpec((HID, HID), full),
            pl.BlockSpec((1, HID), full),
        ],
        out_specs=pl.BlockSpec((G, HID), full),
        out_shape=jax.ShapeDtypeStruct((G, HID), jnp.float32),
    )(pooled, p['cls_w1'], p['cls_b1'].reshape(1, -1),
      p['cls_ln_g'].reshape(1, -1), p['cls_ln_b'].reshape(1, -1),
      w2, b2.reshape(1, -1), w3, b3.reshape(1, -1))
    return out[:, :2]


def kernel(x, params, edge_index, batch):
    p = params
    n = x.shape[0]
    e = edge_index.shape[1]

    e_chunks = -(-e // (NW * CHUNK))
    e_pad = e_chunks * NW * CHUNK
    rpt = (-(-(n + 1) // NSUB) + 7) // 8 * 8
    n_pad = rpt * NSUB
    b_chunks = -(-n // (NW * 64))
    nb_pad = b_chunks * NW * 64

    src = jnp.concatenate(
        [edge_index[0], jnp.zeros((e_pad - e,), jnp.int32)])
    dst = jnp.concatenate(
        [edge_index[1], jnp.full((e_pad - e,), n, jnp.int32)])
    batch_pad = jnp.concatenate(
        [batch, jnp.full((nb_pad - n,), G, jnp.int32)])
    zeros_acc = jnp.zeros((n_pad, HID), jnp.float32)

    deg, cnt = _sc_degrees(dst, batch_pad, n_pad, rpt, e_chunks, b_chunks)
    counts = (cnt[0, :G, 0] + cnt[1, :G, 0]).astype(jnp.int32)
    offs = jnp.concatenate(
        [jnp.zeros((1,), jnp.int32), jnp.cumsum(counts)[:-1]])

    h, mp = _tc_input(x, p, deg, n)
    for i in range(LAYERS):
        agg = _sc_aggregate(mp, src, dst, zeros_acc, n_pad, rpt, e_chunks)
        last = i == LAYERS - 1
        wn = p['attn_in_w'] if last else p['conv_w'][i + 1]
        bn = p['attn_in_b'] if last else jnp.zeros((HID,), jnp.float32)
        h, mp = _tc_layer(agg, mp, deg, h, p['conv_b'][i],
                          p['conv_ln_g'][i], p['conv_ln_b'][i], wn, bn, n,
                          residual=(i > 0), qkv_mode=last)

    qkv_pad = jnp.concatenate([mp, jnp.zeros((128, 3 * HID), jnp.float32)])
    h_pad = jnp.concatenate([h, jnp.zeros((128, HID), jnp.float32)])
    pooled = _tc_attn(offs, counts, qkv_pad, h_pad, p).reshape(G, 3 * HID)
    return _tc_cls(pooled, p)


# speedup vs baseline: 9.0672x; 1.0016x over previous
import functools

import jax
import jax.numpy as jnp
from jax import lax
from jax.experimental import pallas as pl
from jax.experimental.pallas import tpu as pltpu
from jax.experimental.pallas import tpu_sc as plsc

IN_DIM = 128
HID = 128
HEADS = 8
DH = HID // HEADS
LAYERS = 3
G = 64

NCORE = 2
NSUB = 16
NW = NCORE * NSUB
CHUNK = 128


def _layer_norm(x, g, b, eps=1e-5):
    m = jnp.mean(x, axis=-1, keepdims=True)
    v = jnp.mean((x - m) ** 2, axis=-1, keepdims=True)
    return (x - m) * lax.rsqrt(v + eps) * g + b


def _sc_deg_body(n_pad, rpt, e_chunks, b_chunks,
                 dst_hbm, batch_hbm, ones_hbm, zrow_hbm, zcnt_hbm,
                 deg_out, cnt_out,
                 idx_v, idxb_v, ones_v, shared_deg, shared_cnt):
    c = lax.axis_index("c")
    s = lax.axis_index("s")
    pltpu.sync_copy(zrow_hbm.at[pl.ds(s * rpt, rpt)],
                    shared_deg.at[pl.ds(s * rpt, rpt)])

    @pl.when(s == 0)
    def _():
        pltpu.sync_copy(zcnt_hbm, shared_cnt)

    pltpu.sync_copy(ones_hbm, ones_v)
    plsc.subcore_barrier()

    ebase = (c * NSUB + s) * e_chunks * CHUNK

    def echunk(j, carry):
        pltpu.sync_copy(dst_hbm.at[pl.ds(ebase + j * CHUNK, CHUNK)], idx_v)
        pltpu.sync_copy(ones_v, shared_deg.at[idx_v], add=True)
        return carry

    lax.fori_loop(0, e_chunks, echunk, 0)

    nbase = (c * NSUB + s) * b_chunks * 64

    def bchunk(j, carry):
        pltpu.sync_copy(batch_hbm.at[pl.ds(nbase + j * 64, 64)], idxb_v)
        pltpu.sync_copy(ones_v.at[pl.ds(0, 64)], shared_cnt.at[idxb_v],
                        add=True)
        return carry

    lax.fori_loop(0, b_chunks, bchunk, 0)
    plsc.subcore_barrier()

    pltpu.sync_copy(shared_deg.at[pl.ds(s * rpt, rpt)],
                    deg_out.at[c, pl.ds(s * rpt, rpt)])

    @pl.when(s == 0)
    def _():
        pltpu.sync_copy(shared_cnt, cnt_out.at[c])


def _sc_degrees(dst_pad, batch_pad, n_pad, rpt, e_chunks, b_chunks):
    mesh = plsc.VectorSubcoreMesh(core_axis_name="c", subcore_axis_name="s")
    ones = jnp.ones((CHUNK, 16), jnp.float32)
    zrow = jnp.zeros((n_pad, 16), jnp.float32)
    zcnt = jnp.zeros((G + 16, 16), jnp.float32)
    body = functools.partial(_sc_deg_body, n_pad, rpt, e_chunks, b_chunks)
    return pl.kernel(
        body,
        out_type=[
            jax.ShapeDtypeStruct((NCORE, n_pad, 16), jnp.float32),
            jax.ShapeDtypeStruct((NCORE, G + 16, 16), jnp.float32),
        ],
        mesh=mesh,
        scratch_types=[
            pltpu.VMEM((CHUNK,), jnp.int32),
            pltpu.VMEM((64,), jnp.int32),
            pltpu.VMEM((CHUNK, 16), jnp.float32),
            pltpu.VMEM_SHARED((n_pad, 16), jnp.float32),
            pltpu.VMEM_SHARED((G + 16, 16), jnp.float32),
        ],
    )(dst_pad, batch_pad, ones, zrow, zcnt)


GRP = 8


def _sc_agg_body(n_pad, rpt, n_groups,
                 mp_hbm, idx_hbm, zeros_hbm, agg_out,
                 idx_b, rows0, rows1, semg, semi, sems, shared_acc):
    c = lax.axis_index("c")
    s = lax.axis_index("s")
    wid = c * NSUB + s
    pltpu.sync_copy(zeros_hbm.at[pl.ds(s * rpt, rpt)],
                    shared_acc.at[pl.ds(s * rpt, rpt)])
    rows = (rows0, rows1)
    pltpu.sync_copy(idx_hbm.at[wid, 0], idx_b.at[0])
    pltpu.async_copy(idx_hbm.at[wid, 1], idx_b.at[1], semi)
    plsc.subcore_barrier()
    pltpu.async_copy(mp_hbm.at[idx_b.at[0, 0]], rows0, semg)

    def group(gg, carry):
        for gb in (0, 1):
            g = gg * 2 + gb

            def chunk(kk, carry2):
                for kb in (0, 1):
                    k = kk * 2 + kb
                    j = g * GRP + k
                    @pl.when(j >= 1)
                    def _():
                        pltpu.make_async_copy(
                            rows[1 - kb],
                            shared_acc.at[idx_b.at[gb, GRP + kb]],
                            sems).wait()

                    if kb == 0:
                        @pl.when((kk == 0) & (g >= 1) & (g + 1 < n_groups))
                        def _():
                            pltpu.async_copy(idx_hbm.at[wid, g + 1],
                                             idx_b.at[1 - gb], semi)
                        pltpu.async_copy(mp_hbm.at[idx_b.at[gb, k + 1]],
                                         rows[1], semg)
                    else:
                        @pl.when(kk < GRP // 2 - 1)
                        def _():
                            pltpu.async_copy(
                                mp_hbm.at[idx_b.at[gb, k + 1]], rows[0],
                                semg)

                        @pl.when((kk == GRP // 2 - 1) & (g + 1 < n_groups))
                        def _():
                            pltpu.make_async_copy(idx_hbm.at[wid, g],
                                                  idx_b.at[1 - gb],
                                                  semi).wait()
                            pltpu.async_copy(
                                mp_hbm.at[idx_b.at[1 - gb, 0]], rows[0],
                                semg)

                    pltpu.make_async_copy(mp_hbm.at[idx_b.at[gb, k]],
                                          rows[kb], semg).wait()
                    pltpu.async_copy(rows[kb],
                                     shared_acc.at[idx_b.at[gb, GRP + kb]],
                                     sems, add=True)
                return carry2

            lax.fori_loop(0, GRP // 2, chunk, 0)
        return carry

    lax.fori_loop(0, n_groups // 2, group, 0)
    pltpu.make_async_copy(rows[1], shared_acc.at[idx_b.at[0, GRP]],
                          sems).wait()
    plsc.subcore_barrier()
    pltpu.sync_copy(shared_acc.at[pl.ds(s * rpt, rpt)],
                    agg_out.at[c, pl.ds(s * rpt, rpt)])


def _sc_aggregate(mp, idx_comb, zeros_hbm, n_pad, rpt, n_groups):
    mesh = plsc.VectorSubcoreMesh(core_axis_name="c", subcore_axis_name="s")
    body = functools.partial(_sc_agg_body, n_pad, rpt, n_groups)
    return pl.kernel(
        body,
        out_type=jax.ShapeDtypeStruct((NCORE, n_pad, HID), jnp.float32),
        mesh=mesh,
        scratch_types=[
            pltpu.VMEM((2, 2 * GRP, CHUNK), jnp.int32),
            pltpu.VMEM((CHUNK, HID), jnp.float32),
            pltpu.VMEM((CHUNK, HID), jnp.float32),
            pltpu.SemaphoreType.DMA,
            pltpu.SemaphoreType.DMA,
            pltpu.SemaphoreType.DMA,
            pltpu.VMEM_SHARED((n_pad, HID), jnp.float32),
        ],
    )(mp, idx_comb, zeros_hbm)


ROWS = 1000


def _tc_input_body(x_r, w_r, b_r, g_r, be_r, deg_r, w0_r, h0_r, mp_r):
    h = jnp.dot(x_r[...], w_r[...].T, preferred_element_type=jnp.float32)
    h = _layer_norm(h + b_r[...], g_r[...], be_r[...])
    h = jnp.maximum(h, 0.0)
    h0_r[...] = h
    deg = deg_r[0, :, 0:1] + deg_r[1, :, 0:1] + 1.0
    dinv = lax.rsqrt(deg)
    mp_r[...] = jnp.dot(h, w0_r[...].T,
                        preferred_element_type=jnp.float32) * dinv


def _tc_input(x, p, deg, n):
    grid = (n // ROWS,)
    return pl.pallas_call(
        _tc_input_body,
        grid=grid,
        in_specs=[
            pl.BlockSpec((ROWS, IN_DIM), lambda i: (i, 0)),
            pl.BlockSpec((HID, IN_DIM), lambda i: (0, 0)),
            pl.BlockSpec((1, HID), lambda i: (0, 0)),
            pl.BlockSpec((1, HID), lambda i: (0, 0)),
            pl.BlockSpec((1, HID), lambda i: (0, 0)),
            pl.BlockSpec((NCORE, ROWS, 16), lambda i: (0, i, 0)),
            pl.BlockSpec((HID, HID), lambda i: (0, 0)),
        ],
        out_specs=[
            pl.BlockSpec((ROWS, HID), lambda i: (i, 0)),
            pl.BlockSpec((ROWS, HID), lambda i: (i, 0)),
        ],
        out_shape=[
            jax.ShapeDtypeStruct((n, HID), jnp.float32),
            jax.ShapeDtypeStruct((n, HID), jnp.float32),
        ],
    )(x, p['in_w'], p['in_b'].reshape(1, -1), p['in_ln_g'].reshape(1, -1),
      p['in_ln_b'].reshape(1, -1), deg, p['conv_w'][0])


def _tc_layer_body(residual, qkv_mode,
                   agg_r, mp_r, deg_r, hres_r, b_r, g_r, be_r, wn_r, bn_r,
                   hout_r, mout_r):
    deg = deg_r[0, :, 0:1] + deg_r[1, :, 0:1] + 1.0
    dinv = lax.rsqrt(deg)
    pre = (agg_r[0] + agg_r[1] + mp_r[...]) * dinv + b_r[...]
    hh = jnp.maximum(_layer_norm(pre, g_r[...], be_r[...]), 0.0)
    if residual:
        hh = hh + hres_r[...]
    hout_r[...] = hh
    z = jnp.dot(hh, wn_r[...].T, preferred_element_type=jnp.float32)
    if qkv_mode:
        mout_r[...] = z + bn_r[...]
    else:
        mout_r[...] = z * dinv


def _tc_layer(agg, mp, deg, hres, b, g, be, wn, bn, n, residual, qkv_mode):
    grid = (n // ROWS,)
    wdim = wn.shape[0]
    body = functools.partial(_tc_layer_body, residual, qkv_mode)
    return pl.pallas_call(
        body,
        grid=grid,
        in_specs=[
            pl.BlockSpec((NCORE, ROWS, HID), lambda i: (0, i, 0)),
            pl.BlockSpec((ROWS, HID), lambda i: (i, 0)),
            pl.BlockSpec((NCORE, ROWS, 16), lambda i: (0, i, 0)),
            pl.BlockSpec((ROWS, HID), lambda i: (i, 0)),
            pl.BlockSpec((1, HID), lambda i: (0, 0)),
            pl.BlockSpec((1, HID), lambda i: (0, 0)),
            pl.BlockSpec((1, HID), lambda i: (0, 0)),
            pl.BlockSpec((wdim, HID), lambda i: (0, 0)),
            pl.BlockSpec((1, wdim), lambda i: (0, 0)),
        ],
        out_specs=[
            pl.BlockSpec((ROWS, HID), lambda i: (i, 0)),
            pl.BlockSpec((ROWS, wdim), lambda i: (i, 0)),
        ],
        out_shape=[
            jax.ShapeDtypeStruct((n, HID), jnp.float32),
            jax.ShapeDtypeStruct((n, wdim), jnp.float32),
        ],
    )(agg, mp, deg, hres, b.reshape(1, -1), g.reshape(1, -1),
      be.reshape(1, -1), wn, bn.reshape(1, -1))


def _tc_attn_body(offs_r, cnt_r, qkv_r, h_r, wo_r, bo_r, g_r, be_r, out_r):
    gi = pl.program_id(0)
    off = offs_r[gi]
    n = cnt_r[gi]
    off_al = (off // 8) * 8
    d = off - off_al
    nt = (n + d + 127) // 128
    NEG = jnp.float32(-1e30)
    scale = jnp.float32(1.0 / (DH ** 0.5))

    def row_step(rt, row_carry):
        sum_acc, max_acc = row_carry
        r0 = off_al + rt * 128
        q = qkv_r[pl.ds(r0, 128), 0:HID] * scale

        def col_step(ct, cc):
            m_c, l_c, o_c = cc
            c0 = off_al + ct * 128
            k = qkv_r[pl.ds(c0, 128), HID:2 * HID]
            v = qkv_r[pl.ds(c0, 128), 2 * HID:3 * HID]
            colid = ct * 128 + lax.broadcasted_iota(jnp.int32, (1, 128), 1)
            cmask = (colid >= d) & (colid < n + d)
            m_cols, l_cols, o_cols = [], [], []
            for h in range(HEADS):
                sl = slice(h * DH, (h + 1) * DH)
                sc = lax.dot_general(q[:, sl], k[:, sl],
                                     (((1,), (1,)), ((), ())),
                                     preferred_element_type=jnp.float32)
                sc = jnp.where(cmask, sc, NEG)
                m_old = m_c[:, h:h + 1]
                m_new = jnp.maximum(m_old, jnp.max(sc, axis=1, keepdims=True))
                pexp = jnp.exp(sc - m_new)
                alpha = jnp.exp(m_old - m_new)
                l_cols.append(l_c[:, h:h + 1] * alpha
                              + jnp.sum(pexp, axis=1, keepdims=True))
                o_cols.append(o_c[:, sl] * alpha
                              + jnp.dot(pexp, v[:, sl],
                                        preferred_element_type=jnp.float32))
                m_cols.append(m_new)
            return (jnp.concatenate(m_cols, axis=1),
                    jnp.concatenate(l_cols, axis=1),
                    jnp.concatenate(o_cols, axis=1))

        m0 = jnp.full((128, HEADS), NEG, jnp.float32)
        l0 = jnp.zeros((128, HEADS), jnp.float32)
        o0 = jnp.zeros((128, HID), jnp.float32)
        m_c, l_c, o_c = lax.fori_loop(0, nt, col_step, (m0, l0, o0))
        o = jnp.concatenate(
            [o_c[:, h * DH:(h + 1) * DH]
             / jnp.maximum(l_c[:, h:h + 1], jnp.float32(1e-30))
             for h in range(HEADS)], axis=1)
        a = jnp.dot(o, wo_r[...].T,
                    preferred_element_type=jnp.float32) + bo_r[...]
        y = _layer_norm(a + h_r[pl.ds(r0, 128), :], g_r[...], be_r[...])
        rowid = rt * 128 + lax.broadcasted_iota(jnp.int32, (128, 1), 0)
        rmask = (rowid >= d) & (rowid < n + d)
        sum_acc = sum_acc + jnp.sum(jnp.where(rmask, y, 0.0), axis=0,
                                    keepdims=True)
        max_acc = jnp.maximum(
            max_acc,
            jnp.max(jnp.where(rmask, y, -jnp.inf), axis=0, keepdims=True))
        return (sum_acc, max_acc)

    s0 = jnp.zeros((1, HID), jnp.float32)
    mx0 = jnp.full((1, HID), -jnp.inf, jnp.float32)
    sum_acc, max_acc = lax.fori_loop(0, nt, row_step, (s0, mx0))
    mean = sum_acc / jnp.maximum(n.astype(jnp.float32), 1.0)
    out_r[...] = jnp.concatenate([mean, max_acc, sum_acc],
                                 axis=1).reshape(1, 1, 3 * HID)


def _tc_attn(offs, cnt, qkv_pad, h_pad, p):
    npad = qkv_pad.shape[0]
    return pl.pallas_call(
        _tc_attn_body,
        grid=(G,),
        in_specs=[
            pl.BlockSpec(memory_space=pltpu.SMEM),
            pl.BlockSpec(memory_space=pltpu.SMEM),
            pl.BlockSpec((npad, 3 * HID), lambda i: (0, 0)),
            pl.BlockSpec((npad, HID), lambda i: (0, 0)),
            pl.BlockSpec((HID, HID), lambda i: (0, 0)),
            pl.BlockSpec((1, HID), lambda i: (0, 0)),
            pl.BlockSpec((1, HID), lambda i: (0, 0)),
            pl.BlockSpec((1, HID), lambda i: (0, 0)),
        ],
        out_specs=pl.BlockSpec((1, 1, 3 * HID), lambda i: (i, 0, 0)),
        out_shape=jax.ShapeDtypeStruct((G, 1, 3 * HID), jnp.float32),
    )(offs, cnt, qkv_pad, h_pad, p['attn_out_w'],
      p['attn_out_b'].reshape(1, -1), p['attn_ln_g'].reshape(1, -1),
      p['attn_ln_b'].reshape(1, -1))


def _tc_cls_body(x_r, w1_r, b1_r, g_r, be_r, w2_r, b2_r, w3_r, b3_r, out_r):
    z = jnp.dot(x_r[...], w1_r[...].T,
                preferred_element_type=jnp.float32) + b1_r[...]
    z = jnp.maximum(_layer_norm(z, g_r[...], be_r[...]), 0.0)
    z = jnp.maximum(jnp.dot(z, w2_r[...].T,
                            preferred_element_type=jnp.float32) + b2_r[...],
                    0.0)
    out_r[...] = jnp.dot(z, w3_r[...].T,
                         preferred_element_type=jnp.float32) + b3_r[...]


def _tc_cls(pooled, p):
    w2 = jnp.zeros((HID, HID), jnp.float32).at[:HID // 2, :].set(p['cls_w2'])
    b2 = jnp.zeros((HID,), jnp.float32).at[:HID // 2].set(p['cls_b2'])
    w3 = jnp.zeros((HID, HID), jnp.float32).at[:2, :HID // 2].set(p['cls_w3'])
    b3 = jnp.zeros((HID,), jnp.float32).at[:2].set(p['cls_b3'])
    full = lambda i: (0, 0)
    out = pl.pallas_call(
        _tc_cls_body,
        grid=(1,),
        in_specs=[
            pl.BlockSpec((G, 3 * HID), full),
            pl.BlockSpec((HID, 3 * HID), full),
            pl.BlockSpec((1, HID), full),
            pl.BlockSpec((1, HID), full),
            pl.BlockSpec((1, HID), full),
            pl.BlockSpec((HID, HID), full),
            pl.BlockSpec((1, HID), full),
            pl.BlockSpec((HID, HID), full),
            pl.BlockSpec((1, HID), full),
        ],
        out_specs=pl.BlockSpec((G, HID), full),
        out_shape=jax.ShapeDtypeStruct((G, HID), jnp.float32),
    )(pooled, p['cls_w1'], p['cls_b1'].reshape(1, -1),
      p['cls_ln_g'].reshape(1, -1), p['cls_ln_b'].reshape(1, -1),
      w2, b2.reshape(1, -1), w3, b3.reshape(1, -1))
    return out[:, :2]


def kernel(x, params, edge_index, batch):
    p = params
    n = x.shape[0]
    e = edge_index.shape[1]

    e_chunks = 2 * GRP * -(-e // (NW * CHUNK * 2 * GRP))
    n_groups = e_chunks // GRP
    e_pad = e_chunks * NW * CHUNK
    rpt = (-(-(n + 1) // NSUB) + 7) // 8 * 8
    n_pad = rpt * NSUB
    b_chunks = -(-n // (NW * 64))
    nb_pad = b_chunks * NW * 64

    src = jnp.concatenate(
        [edge_index[0], jnp.zeros((e_pad - e,), jnp.int32)])
    dst = jnp.concatenate(
        [edge_index[1], jnp.full((e_pad - e,), n, jnp.int32)])
    batch_pad = jnp.concatenate(
        [batch, jnp.full((nb_pad - n,), G, jnp.int32)])
    zeros_acc = jnp.zeros((n_pad, HID), jnp.float32)

    deg, cnt = _sc_degrees(dst, batch_pad, n_pad, rpt, e_chunks, b_chunks)
    counts = (cnt[0, :G, 0] + cnt[1, :G, 0]).astype(jnp.int32)
    offs = jnp.concatenate(
        [jnp.zeros((1,), jnp.int32), jnp.cumsum(counts)[:-1]])

    idx_comb = jnp.concatenate(
        [src.reshape(NW, n_groups, GRP, CHUNK),
         dst.reshape(NW, n_groups, GRP, CHUNK)], axis=2)

    h, mp = _tc_input(x, p, deg, n)
    for i in range(LAYERS):
        agg = _sc_aggregate(mp, idx_comb, zeros_acc, n_pad, rpt, n_groups)
        last = i == LAYERS - 1
        wn = p['attn_in_w'] if last else p['conv_w'][i + 1]
        bn = p['attn_in_b'] if last else jnp.zeros((HID,), jnp.float32)
        h, mp = _tc_layer(agg, mp, deg, h, p['conv_b'][i],
                          p['conv_ln_g'][i], p['conv_ln_b'][i], wn, bn, n,
                          residual=(i > 0), qkv_mode=last)

    qkv_pad = jnp.concatenate([mp, jnp.zeros((128, 3 * HID), jnp.float32)])
    h_pad = jnp.concatenate([h, jnp.zeros((128, HID), jnp.float32)])
    pooled = _tc_attn(offs, counts, qkv_pad, h_pad, p).reshape(G, 3 * HID)
    return _tc_cls(pooled, p)
